# Initial kernel scaffold; baseline (speedup 1.0000x reference)
import jax
import jax.numpy as jnp
from jax.experimental import pallas as pl


def kernel(inpt, emb_mtrx):
    raise NotImplementedError("write your pallas kernel here")


# speedup vs baseline: 1.3432x; 1.3432x over previous
import functools

import jax
import jax.numpy as jnp
from jax import lax
from jax.experimental import pallas as pl
from jax.experimental.pallas import tpu as pltpu
from jax.experimental.pallas import tpu_sc as plsc

N = 32768
D = 64
K = 1024
TN = 512
STEPS = N // TN

NW = 32
BPW = N // NW
CHUNK = 128
ROWS = BPW // CHUNK


def _distance_body(x_ref, e_ref, idx_ref, loss_ref, perp_ref, cnt_ref, acc_ref):
    step = pl.program_id(0)
    x = x_ref[...]
    e = e_ref[...]
    s = jnp.dot(x, e, preferred_element_type=jnp.float32)
    e2 = jnp.sum(e * e, axis=0, keepdims=True)
    half = e2 - 2.0 * s
    idx = jnp.argmin(half, axis=1).astype(jnp.int32)
    m = jnp.min(half, axis=1)
    x2 = jnp.sum(x * x, axis=1)
    part = jnp.sum(x2 + m)

    iota = lax.broadcasted_iota(jnp.int32, (TN, K), 1)
    onehot = jnp.where(iota == idx[:, None], 1.0, 0.0)
    cnt_step = jnp.sum(onehot, axis=0)

    @pl.when(step == 0)
    def _init():
        cnt_ref[...] = jnp.zeros_like(cnt_ref)
        acc_ref[0] = 0.0

    cnt_ref[...] += cnt_step
    acc_ref[0] += part
    idx_ref[...] = idx

    @pl.when(step == STEPS - 1)
    def _finish():
        loss_ref[...] = jnp.full((1, 1), acc_ref[0] / (N * D), jnp.float32)
        p = cnt_ref[...] * (1.0 / N)
        ent = jnp.sum(p * jnp.log(p + 1e-10))
        perp_ref[...] = jnp.full((1, 1), jnp.exp(-ent), jnp.float32)


def _distance_call(x, e):
    return pl.pallas_call(
        _distance_body,
        grid=(STEPS,),
        in_specs=[
            pl.BlockSpec((TN, D), lambda i: (i, 0)),
            pl.BlockSpec((D, K), lambda i: (0, 0)),
        ],
        out_specs=[
            pl.BlockSpec((TN,), lambda i: (i,)),
            pl.BlockSpec((1, 1), lambda i: (0, 0)),
            pl.BlockSpec((1, 1), lambda i: (0, 0)),
        ],
        out_shape=[
            jax.ShapeDtypeStruct((N,), jnp.int32),
            jax.ShapeDtypeStruct((1, 1), jnp.float32),
            jax.ShapeDtypeStruct((1, 1), jnp.float32),
        ],
        scratch_shapes=[
            pltpu.VMEM((K,), jnp.float32),
            pltpu.SMEM((1,), jnp.float32),
        ],
    )(x, e)


def _make_sc_gather():
    mesh = plsc.VectorSubcoreMesh(core_axis_name="c", subcore_axis_name="s")

    @functools.partial(
        pl.kernel,
        mesh=mesh,
        compiler_params=pltpu.CompilerParams(use_tc_tiling_on_sc=False),
        out_type=jax.ShapeDtypeStruct((N, D), jnp.float32),
        scratch_types=[
            pltpu.VMEM((ROWS, CHUNK), jnp.int32),
            pltpu.VMEM((BPW, D), jnp.float32),
            pltpu.SemaphoreType.DMA,
        ],
    )
    def gather_kernel(table_hbm, idx_hbm, out_hbm, idx_v, rows_v, sem):
        wid = lax.axis_index("s") * 2 + lax.axis_index("c")
        pltpu.sync_copy(idx_hbm.at[pl.ds(wid * ROWS, ROWS)], idx_v)
        copies = []
        for j in range(ROWS):
            copies.append(
                pltpu.async_copy(
                    table_hbm.at[idx_v.at[j]],
                    rows_v.at[pl.ds(j * CHUNK, CHUNK)],
                    sem,
                )
            )
        for c in copies:
            c.wait()
        pltpu.sync_copy(rows_v, out_hbm.at[pl.ds(wid * BPW, BPW)])

    return gather_kernel


_sc_gather_cache = []


def _sc_gather(table, idx2):
    if not _sc_gather_cache:
        _sc_gather_cache.append(_make_sc_gather())
    return _sc_gather_cache[0](table, idx2)


def kernel(inpt, emb_mtrx):
    x = inpt.reshape(N, D)
    idx, loss, perp = _distance_call(x, emb_mtrx)
    table = emb_mtrx.T
    q = _sc_gather(table, idx.reshape(N // CHUNK, CHUNK))
    q = q.reshape(inpt.shape)
    return (q, loss[0, 0], perp[0, 0])


# speedup vs baseline: 1.8551x; 1.3812x over previous
import functools

import jax
import jax.numpy as jnp
from jax import lax
from jax.experimental import pallas as pl
from jax.experimental.pallas import tpu as pltpu
from jax.experimental.pallas import tpu_sc as plsc

N = 32768
D = 64
K = 1024
TN = 4096
STEPS = N // TN

NW = 32
BPW = N // NW
CHUNK = 128
ROWS = BPW // CHUNK


def _distance_body(x_ref, e_ref, idx_ref, loss_ref, perp_ref, cnt_ref, acc_ref):
    step = pl.program_id(0)
    x = x_ref[...]
    e = e_ref[...]
    s = jnp.dot(-2.0 * x, e, preferred_element_type=jnp.float32)
    e2 = jnp.sum(e * e, axis=0, keepdims=True)
    half = s + e2
    m = jnp.min(half, axis=1, keepdims=True)
    ohm = half == m
    iota = lax.broadcasted_iota(jnp.int32, (TN, K), 1).astype(jnp.float32)
    idx = jnp.min(jnp.where(ohm, iota, float(K)), axis=1).astype(jnp.int32)
    oh = jnp.where(ohm, 1.0, 0.0)
    cnt_step = jnp.dot(jnp.full((1, TN), 1.0, jnp.float32), oh,
                       preferred_element_type=jnp.float32)
    part = jnp.sum(x * x) + jnp.sum(m)

    @pl.when(step == 0)
    def _init():
        cnt_ref[...] = jnp.zeros_like(cnt_ref)
        acc_ref[0] = 0.0

    cnt_ref[...] += cnt_step
    acc_ref[0] += part
    idx_ref[...] = idx

    @pl.when(step == STEPS - 1)
    def _finish():
        loss_ref[...] = jnp.full((1, 1), acc_ref[0] / (N * D), jnp.float32)
        p = cnt_ref[...] * (1.0 / N)
        ent = jnp.sum(p * jnp.log(p + 1e-10))
        perp_ref[...] = jnp.full((1, 1), jnp.exp(-ent), jnp.float32)


def _distance_call(x, e):
    return pl.pallas_call(
        _distance_body,
        grid=(STEPS,),
        in_specs=[
            pl.BlockSpec((TN, D), lambda i: (i, 0)),
            pl.BlockSpec((D, K), lambda i: (0, 0)),
        ],
        out_specs=[
            pl.BlockSpec((TN,), lambda i: (i,)),
            pl.BlockSpec((1, 1), lambda i: (0, 0)),
            pl.BlockSpec((1, 1), lambda i: (0, 0)),
        ],
        out_shape=[
            jax.ShapeDtypeStruct((N,), jnp.int32),
            jax.ShapeDtypeStruct((1, 1), jnp.float32),
            jax.ShapeDtypeStruct((1, 1), jnp.float32),
        ],
        scratch_shapes=[
            pltpu.VMEM((1, K), jnp.float32),
            pltpu.SMEM((1,), jnp.float32),
        ],
    )(x, e)


def _make_sc_gather():
    mesh = plsc.VectorSubcoreMesh(core_axis_name="c", subcore_axis_name="s")

    @functools.partial(
        pl.kernel,
        mesh=mesh,
        compiler_params=pltpu.CompilerParams(use_tc_tiling_on_sc=False),
        out_type=jax.ShapeDtypeStruct((N, D), jnp.float32),
        scratch_types=[
            pltpu.VMEM((ROWS, CHUNK), jnp.int32),
            pltpu.VMEM((BPW, D), jnp.float32),
            pltpu.SemaphoreType.DMA,
        ],
    )
    def gather_kernel(table_hbm, idx_hbm, out_hbm, idx_v, rows_v, sem):
        wid = lax.axis_index("s") * 2 + lax.axis_index("c")
        pltpu.sync_copy(idx_hbm.at[pl.ds(wid * ROWS, ROWS)], idx_v)
        copies = []
        for j in range(ROWS):
            copies.append(
                pltpu.async_copy(
                    table_hbm.at[idx_v.at[j]],
                    rows_v.at[pl.ds(j * CHUNK, CHUNK)],
                    sem,
                )
            )
        for c in copies:
            c.wait()
        pltpu.sync_copy(rows_v, out_hbm.at[pl.ds(wid * BPW, BPW)])

    return gather_kernel


_sc_gather_cache = []


def _sc_gather(table, idx2):
    if not _sc_gather_cache:
        _sc_gather_cache.append(_make_sc_gather())
    return _sc_gather_cache[0](table, idx2)


def kernel(inpt, emb_mtrx):
    x = inpt.reshape(N, D)
    idx, loss, perp = _distance_call(x, emb_mtrx)
    table = emb_mtrx.T
    q = _sc_gather(table, idx.reshape(N // CHUNK, CHUNK))
    q = q.reshape(inpt.shape)
    return (q, loss[0, 0], perp[0, 0])
